# baseline (device time: 322031 ns/iter reference)
import jax
import jax.numpy as jnp
from jax import lax
from jax.experimental import pallas as pl
from jax.experimental.pallas import tpu as pltpu

N_DEV = 4
GELU_C = 0.7978845608028654


def kernel(x, w_mat):
    m_glob, k_shard = x.shape
    k_glob, n_out = w_mat.shape
    m_blk = m_glob // N_DEV
    sub_m = m_blk // 4
    n_tile = 256
    n_tiles = n_out // n_tile

    x = x.astype(jnp.bfloat16)

    def body(x_ref, w_ref, out_ref, x_sub, acc, w_vmem,
             send_sems, recv_sems, load_sems, w_sems, out_sem):
        me = lax.axis_index("i")

        barrier_sem = pltpu.get_barrier_semaphore()
        for s in range(1, N_DEV):
            peer = (me + s) % N_DEV
            pl.semaphore_signal(
                barrier_sem, inc=1,
                device_id=(peer,), device_id_type=pl.DeviceIdType.MESH,
            )
        pl.semaphore_wait(barrier_sem, N_DEV - 1)

        local_cps = []
        for q in range(4):
            cp = pltpu.make_async_copy(
                x_ref.at[pl.ds(me * m_blk + q * sub_m, sub_m), :],
                x_sub.at[q], load_sems.at[q])
            cp.start()
            local_cps.append(cp)

        rdmas = {}
        for q in range(4):
            for s in (1, 3, 2):
                r = q * 3 + (s - 1)
                dst = (me + s) % N_DEV
                rd = pltpu.make_async_remote_copy(
                    src_ref=x_ref.at[pl.ds(dst * m_blk + q * sub_m, sub_m), :],
                    dst_ref=x_sub.at[4 + r],
                    send_sem=send_sems.at[r],
                    recv_sem=recv_sems.at[r],
                    device_id=(dst,),
                    device_id_type=pl.DeviceIdType.MESH,
                )
                rd.start()
                rdmas[(s, q)] = rd

        out_cp = None
        for q in range(4):
            for step, kind in enumerate(("local", 1, 3, 2)):
                if kind == "local":
                    local_cps[q].wait()
                    slot = q
                    src = me
                else:
                    rdmas[(kind, q)].wait_recv()
                    slot = 4 + q * 3 + (kind - 1)
                    src = (me - kind) % N_DEV
                first = step == 0
                last = step == 3
                w_row = src * k_shard

                if first and q > 0:
                    out_cp.wait()

                def make_wcp(n):
                    return pltpu.make_async_copy(
                        w_ref.at[pl.ds(w_row, k_shard),
                                 pl.ds(n * n_tile, n_tile)],
                        w_vmem.at[n % 4], w_sems.at[n % 4])

                make_wcp(0).start()
                make_wcp(1).start()
                make_wcp(2).start()

                def tile_step(n, _, slot=slot, first=first, last=last,
                              make_wcp=make_wcp):
                    make_wcp(n).wait()

                    @pl.when(n + 3 < n_tiles)
                    def _():
                        make_wcp(n + 3).start()

                    wb = w_vmem[lax.rem(n, 4)].astype(jnp.bfloat16)
                    part = jnp.dot(x_sub[slot], wb,
                                   preferred_element_type=jnp.float32)
                    sl = pl.ds(n * n_tile, n_tile)
                    if first:
                        acc[:, sl] = part
                    elif not last:
                        acc[:, sl] = acc[:, sl] + part
                    else:
                        y = acc[:, sl] + part
                        acc[:, sl] = 0.5 * y * (1.0 + jnp.tanh(
                            GELU_C * (y + 0.044715 * y * y * y)))
                    return 0

                lax.fori_loop(0, n_tiles, tile_step, 0)

                if last:
                    out_cp = pltpu.make_async_copy(
                        acc, out_ref.at[pl.ds(q * sub_m, sub_m), :], out_sem)
                    out_cp.start()

        out_cp.wait()
        for rd in rdmas.values():
            rd.wait_send()

    return pl.pallas_call(
        body,
        out_shape=jax.ShapeDtypeStruct((m_blk, n_out), jnp.float32),
        in_specs=[
            pl.BlockSpec(memory_space=pl.ANY),
            pl.BlockSpec(memory_space=pl.ANY),
        ],
        out_specs=pl.BlockSpec(memory_space=pl.ANY),
        scratch_shapes=[
            pltpu.VMEM((16, sub_m, k_shard), jnp.bfloat16),
            pltpu.VMEM((sub_m, n_out), jnp.float32),
            pltpu.VMEM((4, k_shard, n_tile), jnp.float32),
            pltpu.SemaphoreType.DMA((12,)),
            pltpu.SemaphoreType.DMA((12,)),
            pltpu.SemaphoreType.DMA((4,)),
            pltpu.SemaphoreType.DMA((4,)),
            pltpu.SemaphoreType.DMA,
        ],
        compiler_params=pltpu.CompilerParams(
            collective_id=0,
            vmem_limit_bytes=64 * 1024 * 1024,
        ),
    )(x, w_mat)


# device time: 294841 ns/iter; 1.0922x vs baseline; 1.0922x over previous
import jax
import jax.numpy as jnp
from jax import lax
from jax.experimental import pallas as pl
from jax.experimental.pallas import tpu as pltpu

N_DEV = 4
GELU_C = 0.7978845608028654


def kernel(x, w_mat):
    m_glob, k_shard = x.shape
    k_glob, n_out = w_mat.shape
    m_blk = m_glob // N_DEV
    sub_m = m_blk // 2
    n_tile = 256
    n_tiles = n_out // n_tile

    x = x.astype(jnp.bfloat16)

    def body(x_ref, w_ref, out_ref, x_sub, acc, w_vmem,
             send_sems, recv_sems, load_sems, w_sems, out_sem):
        me = lax.axis_index("i")

        barrier_sem = pltpu.get_barrier_semaphore()
        for s in range(1, N_DEV):
            peer = (me + s) % N_DEV
            pl.semaphore_signal(
                barrier_sem, inc=1,
                device_id=(peer,), device_id_type=pl.DeviceIdType.MESH,
            )
        pl.semaphore_wait(barrier_sem, N_DEV - 1)

        local_cps = []
        for q in (0, 1):
            cp = pltpu.make_async_copy(
                x_ref.at[pl.ds(me * m_blk + q * sub_m, sub_m), :],
                x_sub.at[q], load_sems.at[q])
            cp.start()
            local_cps.append(cp)

        rdmas = {}
        for q in (0, 1):
            for s in (1, 3, 2):
                r = q * 3 + (s - 1)
                dst = (me + s) % N_DEV
                rd = pltpu.make_async_remote_copy(
                    src_ref=x_ref.at[pl.ds(dst * m_blk + q * sub_m, sub_m), :],
                    dst_ref=x_sub.at[2 + r],
                    send_sem=send_sems.at[r],
                    recv_sem=recv_sems.at[r],
                    device_id=(dst,),
                    device_id_type=pl.DeviceIdType.MESH,
                )
                rd.start()
                rdmas[(s, q)] = rd

        out_cp = None
        for q in (0, 1):
            for step, kind in enumerate(("local", 1, 3, 2)):
                if kind == "local":
                    local_cps[q].wait()
                    slot = q
                    src = me
                else:
                    rdmas[(kind, q)].wait_recv()
                    slot = 2 + q * 3 + (kind - 1)
                    src = (me - kind) % N_DEV
                first = step == 0
                last = step == 3
                w_row = src * k_shard

                if first and q == 1:
                    out_cp.wait()

                def make_wcp(n):
                    return pltpu.make_async_copy(
                        w_ref.at[pl.ds(w_row, k_shard),
                                 pl.ds(n * n_tile, n_tile)],
                        w_vmem.at[n % 4], w_sems.at[n % 4])

                make_wcp(0).start()
                make_wcp(1).start()
                make_wcp(2).start()

                def tile_step(n, _, slot=slot, first=first, last=last,
                              make_wcp=make_wcp):
                    make_wcp(n).wait()

                    @pl.when(n + 3 < n_tiles)
                    def _():
                        make_wcp(n + 3).start()

                    wb = w_vmem[lax.rem(n, 4)].astype(jnp.bfloat16)
                    part = jnp.dot(x_sub[slot], wb,
                                   preferred_element_type=jnp.float32)
                    sl = pl.ds(n * n_tile, n_tile)
                    if first:
                        acc[:, sl] = part
                    elif not last:
                        acc[:, sl] = acc[:, sl] + part
                    else:
                        y = acc[:, sl] + part
                        acc[:, sl] = 0.5 * y * (1.0 + jnp.tanh(
                            GELU_C * (y + 0.044715 * y * y * y)))
                    return 0

                lax.fori_loop(0, n_tiles, tile_step, 0)

                if last:
                    out_cp = pltpu.make_async_copy(
                        acc, out_ref.at[pl.ds(q * sub_m, sub_m), :], out_sem)
                    out_cp.start()

        out_cp.wait()
        for rd in rdmas.values():
            rd.wait_send()

    return pl.pallas_call(
        body,
        out_shape=jax.ShapeDtypeStruct((m_blk, n_out), jnp.float32),
        in_specs=[
            pl.BlockSpec(memory_space=pl.ANY),
            pl.BlockSpec(memory_space=pl.ANY),
        ],
        out_specs=pl.BlockSpec(memory_space=pl.ANY),
        scratch_shapes=[
            pltpu.VMEM((8, sub_m, k_shard), jnp.bfloat16),
            pltpu.VMEM((sub_m, n_out), jnp.float32),
            pltpu.VMEM((4, k_shard, n_tile), jnp.float32),
            pltpu.SemaphoreType.DMA((6,)),
            pltpu.SemaphoreType.DMA((6,)),
            pltpu.SemaphoreType.DMA((2,)),
            pltpu.SemaphoreType.DMA((4,)),
            pltpu.SemaphoreType.DMA,
        ],
        compiler_params=pltpu.CompilerParams(
            collective_id=0,
            vmem_limit_bytes=64 * 1024 * 1024,
        ),
    )(x, w_mat)
